# baseline (speedup 1.0000x reference)
import jax
import jax.numpy as jnp
from jax.experimental import pallas as pl
from jax.experimental.pallas import tpu as pltpu

N = 4096
NFEAT = 4096
NCLASS = 8
BLK = 256


def _fused_kernel(x_ref, adj_ref, wgc_ref, bgc_ref, wlin_ref, blin_ref,
                  ne_ref, y_ref, acc_ref):
    k = pl.program_id(0)
    nb = pl.num_programs(0)

    support_k = jnp.dot(x_ref[...], wgc_ref[...],
                        preferred_element_type=jnp.float32)
    part = jnp.dot(adj_ref[...], support_k,
                   preferred_element_type=jnp.float32)

    @pl.when(k == 0)
    def _init():
        acc_ref[...] = part

    @pl.when(k > 0)
    def _acc():
        acc_ref[...] += part

    @pl.when(k == nb - 1)
    def _epilogue():
        ne = jnp.maximum(acc_ref[...] + bgc_ref[...], 0.0)
        ne_ref[...] = ne
        m = jnp.max(ne, axis=1, keepdims=True)
        ls = ne - m - jnp.log(jnp.sum(jnp.exp(ne - m), axis=1, keepdims=True))
        y_ref[...] = jnp.dot(wlin_ref[...], ls,
                             preferred_element_type=jnp.float32) + blin_ref[...]


@jax.jit
def kernel(x, adj, W_gc, b_gc, W_lin, b_lin):
    nb = NFEAT // BLK
    bgc2 = b_gc.reshape(1, NCLASS)
    blin2 = b_lin.reshape(1, 1)

    ne, y = pl.pallas_call(
        _fused_kernel,
        grid=(nb,),
        in_specs=[
            pl.BlockSpec((BLK, NFEAT), lambda k: (k, 0)),
            pl.BlockSpec((N, BLK), lambda k: (0, k)),
            pl.BlockSpec((NFEAT, NCLASS), lambda k: (0, 0)),
            pl.BlockSpec((1, NCLASS), lambda k: (0, 0)),
            pl.BlockSpec((1, NFEAT), lambda k: (0, 0)),
            pl.BlockSpec((1, 1), lambda k: (0, 0)),
        ],
        out_specs=[
            pl.BlockSpec((N, NCLASS), lambda k: (0, 0)),
            pl.BlockSpec((1, NCLASS), lambda k: (0, 0)),
        ],
        out_shape=[
            jax.ShapeDtypeStruct((N, NCLASS), jnp.float32),
            jax.ShapeDtypeStruct((1, NCLASS), jnp.float32),
        ],
        scratch_shapes=[pltpu.VMEM((N, NCLASS), jnp.float32)],
    )(x, adj, W_gc, bgc2, W_lin, blin2)
    return (y, ne)
